# Initial kernel scaffold; baseline (speedup 1.0000x reference)
import jax
import jax.numpy as jnp
from jax.experimental import pallas as pl


def kernel(h_neigh, h_self, edge_features, W_preagg, W_self, W_neigh, W_edge, b_edge, edge_index):
    raise NotImplementedError("write your pallas kernel here")


# speedup vs baseline: 26.9462x; 26.9462x over previous
import functools

import jax
import jax.numpy as jnp
from jax import lax
from jax.experimental import pallas as pl
from jax.experimental.pallas import tpu as pltpu
from jax.experimental.pallas import tpu_sc as plsc

N = 10000
E = 50000
DIN = 256
DOUT = 32
DE = 16

NC = 2
NS = 16
NW = NC * NS
CHUNK = 128
CH_PER_TILE = 13
E_TILE = CHUNK * CH_PER_TILE
E_PAD = NW * E_TILE
N_PAD = 10240
ROWS_TILE = N_PAD // NS
AW = 48

_SC_MESH = plsc.VectorSubcoreMesh(
    core_axis_name="c", subcore_axis_name="s", num_cores=NC, num_subcores=NS)


@functools.partial(
    pl.kernel,
    out_type=jax.ShapeDtypeStruct((NC, N_PAD, AW), jnp.float32),
    mesh=_SC_MESH,
    compiler_params=pltpu.CompilerParams(use_tc_tiling_on_sc=False),
    scratch_types=[
        pltpu.VMEM((CH_PER_TILE, 1, CHUNK), jnp.int32),
        pltpu.VMEM((CH_PER_TILE, 1, CHUNK), jnp.int32),
        pltpu.VMEM((CHUNK, DOUT), jnp.float32),
        pltpu.VMEM((CHUNK, DOUT), jnp.float32),
        pltpu.VMEM((CHUNK, AW), jnp.float32),
        pltpu.VMEM_SHARED((N_PAD, AW), jnp.float32),
        pltpu.SemaphoreType.DMA,
    ],
)
def _sc_edge_scatter(hn_hbm, src_hbm, dst_hbm, ew_hbm, out_hbm,
                     idx_src, idx_dst, ew_buf, rows_buf, msg_buf, acc_sp, sem):
    c = lax.axis_index("c")
    s = lax.axis_index("s")
    wid = c * NS + s

    zeros16 = jnp.zeros((16,), jnp.float32)

    def _zero_row(i, _):
        msg_buf[i, pl.ds(0, 16)] = zeros16
        msg_buf[i, pl.ds(16, 16)] = zeros16
        msg_buf[i, pl.ds(32, 16)] = zeros16
        return 0
    lax.fori_loop(0, CHUNK, _zero_row, 0)

    def _zero_acc(j, _):
        pltpu.sync_copy(msg_buf, acc_sp.at[pl.ds(s * ROWS_TILE + j * CHUNK, CHUNK)])
        return 0
    lax.fori_loop(0, ROWS_TILE // CHUNK, _zero_acc, 0)
    plsc.subcore_barrier()

    deg_lane = jnp.where(lax.iota(jnp.int32, 16) == 0,
                         jnp.float32(1.0), jnp.float32(0.0))

    def _set_deg(i, _):
        msg_buf[i, pl.ds(DOUT, 16)] = deg_lane
        return 0
    lax.fori_loop(0, CHUNK, _set_deg, 0)

    pltpu.sync_copy(src_hbm.at[pl.ds(wid * CH_PER_TILE, CH_PER_TILE)], idx_src)
    pltpu.sync_copy(dst_hbm.at[pl.ds(wid * CH_PER_TILE, CH_PER_TILE)], idx_dst)

    base_e = wid * E_TILE

    def _chunk(j, _):
        pltpu.sync_copy(ew_hbm.at[pl.ds(base_e + j * CHUNK, CHUNK)], ew_buf)
        pltpu.async_copy(hn_hbm.at[idx_src.at[j, 0]], rows_buf, sem).wait()

        def _mul(i, _):
            msg_buf[i, pl.ds(0, 16)] = rows_buf[i, pl.ds(0, 16)] * ew_buf[i, pl.ds(0, 16)]
            msg_buf[i, pl.ds(16, 16)] = rows_buf[i, pl.ds(16, 16)] * ew_buf[i, pl.ds(16, 16)]
            return 0
        lax.fori_loop(0, CHUNK, _mul, 0)

        pltpu.sync_copy(msg_buf, acc_sp.at[idx_dst.at[j, 0]], add=True)
        return 0
    lax.fori_loop(0, CH_PER_TILE, _chunk, 0)
    plsc.subcore_barrier()

    def _export(j, _):
        r0 = s * ROWS_TILE + j * CHUNK
        pltpu.sync_copy(acc_sp.at[pl.ds(r0, CHUNK)], msg_buf)
        pltpu.sync_copy(msg_buf, out_hbm.at[c, pl.ds(r0, CHUNK)])
        return 0
    lax.fori_loop(0, ROWS_TILE // CHUNK, _export, 0)


def _preagg_body(x_ref, w_ref, o_ref):
    y = lax.dot_general(x_ref[...], w_ref[...], (((1,), (1,)), ((), ())),
                        preferred_element_type=jnp.float32)
    o_ref[...] = jnp.maximum(y, 0.0)


def _edge_body(ef_ref, w_ref, b_ref, o_ref):
    y = lax.dot_general(ef_ref[...], w_ref[...], (((1,), (1,)), ((), ())),
                        preferred_element_type=jnp.float32)
    y = jnp.maximum(y + b_ref[...], 0.0)
    for half in (512, 256, 128, 64, 32):
        y = y[:, :half] + y[:, half:2 * half]
    o_ref[...] = y


def _final_body(hs_ref, acc_ref, wp_ref, ws_ref, wn_ref, o_ref):
    hs = jnp.maximum(
        lax.dot_general(hs_ref[...], wp_ref[...], (((1,), (1,)), ((), ())),
                        preferred_element_type=jnp.float32), 0.0)
    a = acc_ref[0] + acc_ref[1]
    neigh = a[:, :DOUT] / jnp.maximum(a[:, DOUT:DOUT + 1], 1.0)
    z1 = jnp.maximum(
        lax.dot_general(hs, ws_ref[...], (((1,), (1,)), ((), ())),
                        preferred_element_type=jnp.float32), 0.0)
    z2 = jnp.maximum(
        lax.dot_general(neigh, wn_ref[...], (((1,), (1,)), ((), ())),
                        preferred_element_type=jnp.float32), 0.0)
    o_ref[...] = jnp.maximum(z1 + z2, 0.0)


def kernel(h_neigh, h_self, edge_features, W_preagg, W_self, W_neigh,
           W_edge, b_edge, edge_index):
    src = edge_index[0]
    dst = edge_index[1]
    src_pad = jnp.concatenate(
        [src, jnp.zeros((E_PAD - E,), jnp.int32)]).reshape(NW * CH_PER_TILE, 1, CHUNK)
    dst_pad = jnp.concatenate(
        [dst, jnp.full((E_PAD - E,), N_PAD - 1, jnp.int32)]).reshape(NW * CH_PER_TILE, 1, CHUNK)
    ef_pad = jnp.zeros((E_PAD, DE), jnp.float32).at[:E].set(edge_features)

    hn = pl.pallas_call(
        _preagg_body,
        grid=(10,),
        in_specs=[pl.BlockSpec((1000, DIN), lambda i: (i, 0)),
                  pl.BlockSpec((DOUT, DIN), lambda i: (0, 0))],
        out_specs=pl.BlockSpec((1000, DOUT), lambda i: (i, 0)),
        out_shape=jax.ShapeDtypeStruct((N, DOUT), jnp.float32),
    )(h_neigh, W_preagg)

    EB = 512
    ew = pl.pallas_call(
        _edge_body,
        grid=(E_PAD // EB,),
        in_specs=[pl.BlockSpec((EB, DE), lambda i: (i, 0)),
                  pl.BlockSpec((DOUT * DOUT, DE), lambda i: (0, 0)),
                  pl.BlockSpec((1, DOUT * DOUT), lambda i: (0, 0))],
        out_specs=pl.BlockSpec((EB, DOUT), lambda i: (i, 0)),
        out_shape=jax.ShapeDtypeStruct((E_PAD, DOUT), jnp.float32),
    )(ef_pad, W_edge, b_edge.reshape(1, DOUT * DOUT))

    acc = _sc_edge_scatter(hn, src_pad, dst_pad, ew)

    z = pl.pallas_call(
        _final_body,
        grid=(10,),
        in_specs=[pl.BlockSpec((1000, DIN), lambda i: (i, 0)),
                  pl.BlockSpec((NC, 1000, AW), lambda i: (0, i, 0)),
                  pl.BlockSpec((DOUT, DIN), lambda i: (0, 0)),
                  pl.BlockSpec((DOUT, DOUT), lambda i: (0, 0)),
                  pl.BlockSpec((DOUT, DOUT), lambda i: (0, 0))],
        out_specs=pl.BlockSpec((1000, DOUT), lambda i: (i, 0)),
        out_shape=jax.ShapeDtypeStruct((N, DOUT), jnp.float32),
    )(h_self, acc, W_preagg, W_self, W_neigh)
    return z
